# baseline (speedup 1.0000x reference)
import jax
import jax.numpy as jnp
from jax.experimental import pallas as pl
from jax.experimental.pallas import tpu as pltpu

N, K, D = 65536, 1024, 32
BN = 1024
NB = N // BN


def _body(x_ref, c_ref, assign_ref, cent_ref, sdist_ref,
          sums_scr, counts_scr, sacc_scr):
    i = pl.program_id(0)

    @pl.when(i == 0)
    def _init():
        sums_scr[...] = jnp.zeros_like(sums_scr)
        counts_scr[...] = jnp.zeros_like(counts_scr)
        sacc_scr[...] = jnp.zeros_like(sacc_scr)

    x = x_ref[...]
    c = c_ref[...]
    cn = jnp.sum(c * c, axis=1, keepdims=True)
    ca = jnp.concatenate([c, cn], axis=1)
    ones_col = jnp.ones((BN, 1), jnp.float32)
    xa = jnp.concatenate([-2.0 * x, ones_col], axis=1)
    dist = jax.lax.dot_general(
        xa, ca, dimension_numbers=(((1,), (1,)), ((), ())),
        preferred_element_type=jnp.float32,
        precision=jax.lax.Precision.HIGHEST)

    minval = jnp.min(dist, axis=1, keepdims=True)
    iota_k = jax.lax.broadcasted_iota(jnp.int32, (BN, K), 1)
    masked = jnp.where(dist == minval, iota_k, K)
    idx = jnp.min(masked, axis=1, keepdims=True)
    assign_ref[...] = idx

    onehot = (iota_k == idx).astype(jnp.float32)
    xo = jnp.concatenate([x, ones_col], axis=1)
    so = jax.lax.dot_general(
        onehot, xo, dimension_numbers=(((0,), (0,)), ((), ())),
        preferred_element_type=jnp.float32,
        precision=jax.lax.Precision.HIGHEST)
    sums_scr[...] += so[:, :D]
    counts_scr[...] += so[:, D:]
    xn = jnp.sum(x * x, axis=1, keepdims=True)
    sacc_scr[...] = sacc_scr[...] + jnp.sum(minval + xn)

    @pl.when(i == NB - 1)
    def _finish():
        cent_ref[...] = sums_scr[...] / counts_scr[...]
        sdist_ref[...] = sacc_scr[...]


@jax.jit
def kernel(input_x, input_centroids):
    assign2, cent, sdist = pl.pallas_call(
        _body,
        grid=(NB,),
        in_specs=[
            pl.BlockSpec((BN, D), lambda i: (i, 0)),
            pl.BlockSpec((K, D), lambda i: (0, 0)),
        ],
        out_specs=[
            pl.BlockSpec((BN, 1), lambda i: (i, 0)),
            pl.BlockSpec((K, D), lambda i: (0, 0)),
            pl.BlockSpec((1, 1), lambda i: (0, 0)),
        ],
        out_shape=[
            jax.ShapeDtypeStruct((N, 1), jnp.int32),
            jax.ShapeDtypeStruct((K, D), jnp.float32),
            jax.ShapeDtypeStruct((1, 1), jnp.float32),
        ],
        scratch_shapes=[
            pltpu.VMEM((K, D), jnp.float32),
            pltpu.VMEM((K, 1), jnp.float32),
            pltpu.VMEM((1, 1), jnp.float32),
        ],
    )(input_x, input_centroids)
    assignments = assign2.reshape(N)
    return assignments, cent, sdist[0, 0]


# speedup vs baseline: 2.8094x; 2.8094x over previous
import jax
import jax.numpy as jnp
from jax.experimental import pallas as pl
from jax.experimental.pallas import tpu as pltpu

N, K, D = 65536, 1024, 32
BN = 1024
NB = N // BN


def _body(x_ref, c_ref, assign_ref, cent_ref, sdist_ref,
          sums_scr, counts_scr, sacc_scr):
    i = pl.program_id(0)

    @pl.when(i == 0)
    def _init():
        sums_scr[...] = jnp.zeros_like(sums_scr)
        counts_scr[...] = jnp.zeros_like(counts_scr)
        sacc_scr[...] = jnp.zeros_like(sacc_scr)

    x = x_ref[...]
    c = c_ref[...]
    cn = jnp.sum(c * c, axis=1, keepdims=True)
    ca = jnp.concatenate([c, cn], axis=1)
    ones_col = jnp.ones((BN, 1), jnp.float32)
    xa = jnp.concatenate([-2.0 * x, ones_col], axis=1)

    def _split(v):
        hi = v.astype(jnp.bfloat16)
        lo = (v - hi.astype(jnp.float32)).astype(jnp.bfloat16)
        return hi, lo

    xa_hi, xa_lo = _split(xa)
    ca_hi, ca_lo = _split(ca)

    def _dot_t(a, b):
        return jax.lax.dot_general(
            a, b, dimension_numbers=(((1,), (1,)), ((), ())),
            preferred_element_type=jnp.float32)

    dist = (_dot_t(xa_hi, ca_hi) + (_dot_t(xa_hi, ca_lo)
                                    + _dot_t(xa_lo, ca_hi)))

    minval = jnp.min(dist, axis=1, keepdims=True)
    iota_k = jax.lax.broadcasted_iota(jnp.int32, (BN, K), 1)
    masked = jnp.where(dist == minval, iota_k, K)
    idx = jnp.min(masked, axis=1, keepdims=True)
    assign_ref[...] = idx

    onehot = (iota_k == idx).astype(jnp.bfloat16)
    x_hi, x_lo = _split(x)

    def _dot_n(a, b):
        return jax.lax.dot_general(
            a, b, dimension_numbers=(((0,), (0,)), ((), ())),
            preferred_element_type=jnp.float32)

    sums_scr[...] += _dot_n(onehot, x_hi) + _dot_n(onehot, x_lo)
    counts_scr[...] += _dot_n(onehot, ones_col.astype(jnp.bfloat16))
    xn = jnp.sum(x * x, axis=1, keepdims=True)
    sacc_scr[...] = sacc_scr[...] + jnp.sum(minval + xn)

    @pl.when(i == NB - 1)
    def _finish():
        cent_ref[...] = sums_scr[...] / counts_scr[...]
        sdist_ref[...] = sacc_scr[...]


@jax.jit
def kernel(input_x, input_centroids):
    assign2, cent, sdist = pl.pallas_call(
        _body,
        grid=(NB,),
        in_specs=[
            pl.BlockSpec((BN, D), lambda i: (i, 0)),
            pl.BlockSpec((K, D), lambda i: (0, 0)),
        ],
        out_specs=[
            pl.BlockSpec((BN, 1), lambda i: (i, 0)),
            pl.BlockSpec((K, D), lambda i: (0, 0)),
            pl.BlockSpec((1, 1), lambda i: (0, 0)),
        ],
        out_shape=[
            jax.ShapeDtypeStruct((N, 1), jnp.int32),
            jax.ShapeDtypeStruct((K, D), jnp.float32),
            jax.ShapeDtypeStruct((1, 1), jnp.float32),
        ],
        scratch_shapes=[
            pltpu.VMEM((K, D), jnp.float32),
            pltpu.VMEM((K, 1), jnp.float32),
            pltpu.VMEM((1, 1), jnp.float32),
        ],
    )(input_x, input_centroids)
    assignments = assign2.reshape(N)
    return assignments, cent, sdist[0, 0]


# speedup vs baseline: 3.2467x; 1.1556x over previous
import jax
import jax.numpy as jnp
from jax.experimental import pallas as pl
from jax.experimental.pallas import tpu as pltpu

N, K, D = 65536, 1024, 32
BN = 1024
NB = N // BN
DA = D + 1


def _body(x_ref, ct_ref, assign_ref, cent_ref, sdist_ref,
          sums_scr, sacc_scr):
    i = pl.program_id(0)

    @pl.when(i == 0)
    def _init():
        sums_scr[...] = jnp.zeros_like(sums_scr)
        sacc_scr[...] = jnp.zeros_like(sacc_scr)

    x = x_ref[...]
    ct = ct_ref[...]
    cn = jnp.sum(ct * ct, axis=0, keepdims=True)
    ca = jnp.concatenate([-2.0 * ct, cn], axis=0)
    ones_col = jnp.ones((BN, 1), jnp.float32)
    xa = jnp.concatenate([x, ones_col], axis=1)

    def _split(v):
        hi = v.astype(jnp.bfloat16)
        lo = (v - hi.astype(jnp.float32)).astype(jnp.bfloat16)
        return hi, lo

    xa_hi, xa_lo = _split(xa)
    ca_hi, ca_lo = _split(ca)

    def _dot(a, b):
        return jax.lax.dot_general(
            a, b, dimension_numbers=(((1,), (0,)), ((), ())),
            preferred_element_type=jnp.float32)

    dist = _dot(xa_hi, ca_hi) + (_dot(xa_hi, ca_lo)
                                 + _dot(xa_lo, ca_hi))

    minval = jnp.min(dist, axis=1, keepdims=True)
    iota_k = jax.lax.broadcasted_iota(jnp.int32, (BN, K), 1)
    masked = jnp.where(dist == minval, iota_k, K)
    idx = jnp.min(masked, axis=1, keepdims=True)
    assign_ref[...] = idx

    idx_row = jnp.transpose(idx)
    iota_kt = jax.lax.broadcasted_iota(jnp.int32, (K, BN), 0)
    onehot_t = (iota_kt == idx_row).astype(jnp.bfloat16)
    sums_scr[...] += _dot(onehot_t, xa_hi) + _dot(onehot_t, xa_lo)

    xn = jnp.sum(x * x, axis=1, keepdims=True)
    sacc_scr[...] = sacc_scr[...] + jnp.sum(minval + xn)

    @pl.when(i == NB - 1)
    def _finish():
        cent_ref[...] = sums_scr[:, :D] / sums_scr[:, D:]
        sdist_ref[...] = sacc_scr[...]


@jax.jit
def kernel(input_x, input_centroids):
    assign2, cent, sdist = pl.pallas_call(
        _body,
        grid=(NB,),
        in_specs=[
            pl.BlockSpec((BN, D), lambda i: (i, 0)),
            pl.BlockSpec((D, K), lambda i: (0, 0)),
        ],
        out_specs=[
            pl.BlockSpec((BN, 1), lambda i: (i, 0)),
            pl.BlockSpec((K, D), lambda i: (0, 0)),
            pl.BlockSpec((1, 1), lambda i: (0, 0)),
        ],
        out_shape=[
            jax.ShapeDtypeStruct((N, 1), jnp.int32),
            jax.ShapeDtypeStruct((K, D), jnp.float32),
            jax.ShapeDtypeStruct((1, 1), jnp.float32),
        ],
        scratch_shapes=[
            pltpu.VMEM((K, DA), jnp.float32),
            pltpu.VMEM((1, 1), jnp.float32),
        ],
    )(input_x, input_centroids.T)
    assignments = assign2.reshape(N)
    return assignments, cent, sdist[0, 0]


# speedup vs baseline: 4.3372x; 1.3359x over previous
import jax
import jax.numpy as jnp
from jax.experimental import pallas as pl
from jax.experimental.pallas import tpu as pltpu

N, K, D = 65536, 1024, 32
BN = 1024
NB = N // BN
DA = D + 1


def _body(x_ref, ct_ref, assign_ref, cent_ref, sdist_ref,
          sums_scr, sacc_scr):
    i = pl.program_id(0)

    @pl.when(i == 0)
    def _init():
        sums_scr[...] = jnp.zeros_like(sums_scr)
        sacc_scr[...] = jnp.zeros_like(sacc_scr)

    x = x_ref[...]
    ct = ct_ref[...]
    cn = jnp.sum(ct * ct, axis=0, keepdims=True)
    ca = jnp.concatenate([-2.0 * ct, cn], axis=0)
    ones_col = jnp.ones((BN, 1), jnp.float32)
    xa = jnp.concatenate([x, ones_col], axis=1)

    def _split(v):
        hi = v.astype(jnp.bfloat16)
        lo = (v - hi.astype(jnp.float32)).astype(jnp.bfloat16)
        return hi, lo

    xa_hi, xa_lo = _split(xa)
    ca_hi, ca_lo = _split(ca)

    def _dot(a, b):
        return jax.lax.dot_general(
            a, b, dimension_numbers=(((1,), (0,)), ((), ())),
            preferred_element_type=jnp.float32)

    x3 = jnp.concatenate([xa_hi, xa_hi, xa_lo], axis=1)
    c3 = jnp.concatenate([ca_hi, ca_lo, ca_hi], axis=0)
    dist = _dot(x3, c3)

    minval = jnp.min(dist, axis=1, keepdims=True)
    iota_k = jax.lax.broadcasted_iota(jnp.int32, (BN, K), 1)
    masked = jnp.where(dist == minval, iota_k, K)
    idx = jnp.min(masked, axis=1, keepdims=True)
    assign_ref[...] = idx

    idx_row = jnp.transpose(idx)
    iota_kt = jax.lax.broadcasted_iota(jnp.int32, (K, BN), 0)
    onehot_t = (iota_kt == idx_row).astype(jnp.bfloat16)
    xa2 = jnp.concatenate([xa_hi, xa_lo], axis=1)
    s2 = _dot(onehot_t, xa2)
    sums_scr[...] += s2[:, :DA] + s2[:, DA:]

    xn = jnp.sum(x * x, axis=1, keepdims=True)
    sacc_scr[...] = sacc_scr[...] + jnp.sum(minval + xn)

    @pl.when(i == NB - 1)
    def _finish():
        cent_ref[...] = sums_scr[:, :D] / sums_scr[:, D:]
        sdist_ref[...] = sacc_scr[...]


@jax.jit
def kernel(input_x, input_centroids):
    assign2, cent, sdist = pl.pallas_call(
        _body,
        grid=(NB,),
        in_specs=[
            pl.BlockSpec((BN, D), lambda i: (i, 0)),
            pl.BlockSpec((D, K), lambda i: (0, 0)),
        ],
        out_specs=[
            pl.BlockSpec((BN, 1), lambda i: (i, 0)),
            pl.BlockSpec((K, D), lambda i: (0, 0)),
            pl.BlockSpec((1, 1), lambda i: (0, 0)),
        ],
        out_shape=[
            jax.ShapeDtypeStruct((N, 1), jnp.int32),
            jax.ShapeDtypeStruct((K, D), jnp.float32),
            jax.ShapeDtypeStruct((1, 1), jnp.float32),
        ],
        scratch_shapes=[
            pltpu.VMEM((K, DA), jnp.float32),
            pltpu.VMEM((1, 1), jnp.float32),
        ],
    )(input_x, input_centroids.T)
    assignments = assign2.reshape(N)
    return assignments, cent, sdist[0, 0]


# speedup vs baseline: 4.8166x; 1.1105x over previous
import jax
import jax.numpy as jnp
from jax.experimental import pallas as pl
from jax.experimental.pallas import tpu as pltpu

N, K, D = 65536, 1024, 32
BN = 1024
NB = N // BN
DA = D + 1


def _body(x_ref, ct_ref, assign_ref, cent_ref, sdist_ref,
          sums_scr, sacc_scr):
    i = pl.program_id(0)

    @pl.when(i == 0)
    def _init():
        sums_scr[...] = jnp.zeros_like(sums_scr)
        sacc_scr[...] = jnp.zeros_like(sacc_scr)

    x = x_ref[...]
    ct = ct_ref[...]
    cn = jnp.sum(ct * ct, axis=0, keepdims=True)
    ca = jnp.concatenate([-2.0 * ct, cn], axis=0)
    ones_col = jnp.ones((BN, 1), jnp.float32)
    xa = jnp.concatenate([x, ones_col], axis=1)

    def _split(v):
        hi = v.astype(jnp.bfloat16)
        lo = (v - hi.astype(jnp.float32)).astype(jnp.bfloat16)
        return hi, lo

    xa_hi, xa_lo = _split(xa)
    ca_hi, ca_lo = _split(ca)

    def _dot(a, b):
        return jax.lax.dot_general(
            a, b, dimension_numbers=(((1,), (0,)), ((), ())),
            preferred_element_type=jnp.float32)

    x3 = jnp.concatenate([xa_hi, xa_hi, xa_lo], axis=1)
    c3 = jnp.concatenate([ca_hi, ca_lo, ca_hi], axis=0)
    dist = _dot(x3, c3)

    minval = jnp.min(dist, axis=1, keepdims=True)
    iota_kf = jax.lax.broadcasted_iota(
        jnp.int32, (BN, K), 1).astype(jnp.float32)
    masked = jnp.where(dist == minval, iota_kf, jnp.float32(K))
    idx = jnp.min(masked, axis=1, keepdims=True).astype(jnp.int32)
    assign_ref[...] = idx

    idx_row = jnp.transpose(idx)
    iota_kt = jax.lax.broadcasted_iota(jnp.int32, (K, BN), 0)
    onehot_t = (iota_kt == idx_row).astype(jnp.bfloat16)
    xa2 = jnp.concatenate([xa_hi, xa_lo], axis=1)
    s2 = _dot(onehot_t, xa2)
    sums_scr[...] += s2[:, :DA] + s2[:, DA:]

    xn = jnp.sum(x * x, axis=1, keepdims=True)
    sacc_scr[...] = sacc_scr[...] + jnp.sum(minval + xn)

    @pl.when(i == NB - 1)
    def _finish():
        cent_ref[...] = sums_scr[:, :D] / sums_scr[:, D:]
        sdist_ref[...] = sacc_scr[...]


@jax.jit
def kernel(input_x, input_centroids):
    assign2, cent, sdist = pl.pallas_call(
        _body,
        grid=(NB,),
        in_specs=[
            pl.BlockSpec((BN, D), lambda i: (i, 0)),
            pl.BlockSpec((D, K), lambda i: (0, 0)),
        ],
        out_specs=[
            pl.BlockSpec((BN, 1), lambda i: (i, 0)),
            pl.BlockSpec((K, D), lambda i: (0, 0)),
            pl.BlockSpec((1, 1), lambda i: (0, 0)),
        ],
        out_shape=[
            jax.ShapeDtypeStruct((N, 1), jnp.int32),
            jax.ShapeDtypeStruct((K, D), jnp.float32),
            jax.ShapeDtypeStruct((1, 1), jnp.float32),
        ],
        scratch_shapes=[
            pltpu.VMEM((K, DA), jnp.float32),
            pltpu.VMEM((1, 1), jnp.float32),
        ],
    )(input_x, input_centroids.T)
    assignments = assign2.reshape(N)
    return assignments, cent, sdist[0, 0]
